# Initial kernel scaffold; baseline (speedup 1.0000x reference)
import jax
import jax.numpy as jnp
from jax.experimental import pallas as pl


def kernel(features, coords):
    raise NotImplementedError("write your pallas kernel here")


# speedup vs baseline: 1.0535x; 1.0535x over previous
import dataclasses

import jax
import jax.numpy as jnp
from jax import lax
from jax.experimental import pallas as pl
from jax.experimental.pallas import tpu as pltpu
from jax.experimental.pallas import tpu_sc as plsc

R = 32
B = 4
C = 128
N = 100000
V = R * R * R
CH = 10000
NCHUNK = N // CH
UNROLL = 5
NIT = CH // (16 * UNROLL)


def _coords_body(coords_ref, norm_ref, idx_ref):
    c = coords_ref[...]
    norm = jnp.clip(((c + 1.0) / 2.0) * R, 0.0, R - 1.0)
    norm_ref[...] = norm
    v = jnp.round(norm).astype(jnp.int32)
    idx_ref[...] = (v[:, 0] * R + v[:, 1]) * R + v[:, 2]


def _sc_body(feat_hbm, idx_hbm, out_hbm, grid_ref, cnt_ref, idx_buf, val_buf):
    wid = lax.axis_index("s") * 2 + lax.axis_index("c")
    b = wid % B
    g = wid // B
    row0 = b * C + g * 16

    zeros = jnp.zeros((16,), jnp.float32)
    ones = jnp.ones((16,), jnp.float32)

    @pl.loop(0, V // 16)
    def _(i):
        cnt_ref[pl.ds(i * 16, 16)] = zeros

    @pl.loop(0, NCHUNK)
    def _(ci):
        pltpu.sync_copy(idx_hbm.at[pl.ds(b * N + ci * CH, CH)], idx_buf)

        @pl.loop(0, NIT)
        def _(i):
            for u in range(UNROLL):
                iv = idx_buf[pl.ds(i * (16 * UNROLL) + u * 16, 16)]
                plsc.addupdate_scatter(cnt_ref, [iv], ones)

    @pl.loop(0, V // 16)
    def _(i):
        sl = pl.ds(i * 16, 16)
        cnt_ref[sl] = 1.0 / jnp.maximum(cnt_ref[sl], 1.0)

    @pl.loop(0, 16)
    def _(k):
        row = row0 + k

        @pl.loop(0, V // 16)
        def _(i):
            grid_ref[pl.ds(i * 16, 16)] = zeros

        @pl.loop(0, NCHUNK)
        def _(ci):
            pltpu.sync_copy(idx_hbm.at[pl.ds(b * N + ci * CH, CH)], idx_buf)
            pltpu.sync_copy(feat_hbm.at[pl.ds(row * N + ci * CH, CH)], val_buf)

            @pl.loop(0, NIT)
            def _(i):
                for u in range(UNROLL):
                    sl = pl.ds(i * (16 * UNROLL) + u * 16, 16)
                    plsc.addupdate_scatter(grid_ref, [idx_buf[sl]], val_buf[sl])

        @pl.loop(0, V // 16)
        def _(i):
            sl = pl.ds(i * 16, 16)
            grid_ref[sl] = grid_ref[sl] * cnt_ref[sl]

        pltpu.sync_copy(grid_ref, out_hbm.at[pl.ds(row * V, V)])


_cp = pltpu.CompilerParams()
if "needs_layout_passes" in pltpu.CompilerParams.__dataclass_fields__:
    _cp = dataclasses.replace(_cp, needs_layout_passes=False)

_scatter_mean = pl.kernel(
    _sc_body,
    out_type=jax.ShapeDtypeStruct((B * C * V,), jnp.float32),
    mesh=plsc.VectorSubcoreMesh(core_axis_name="c", subcore_axis_name="s"),
    scratch_types=[
        pltpu.VMEM((V,), jnp.float32),
        pltpu.VMEM((V,), jnp.float32),
        pltpu.VMEM((CH,), jnp.int32),
        pltpu.VMEM((CH,), jnp.float32),
    ],
    compiler_params=_cp,
)


def kernel(features, coords):
    norm, idx = pl.pallas_call(
        _coords_body,
        out_shape=[
            jax.ShapeDtypeStruct((B, 3, N), jnp.float32),
            jax.ShapeDtypeStruct((B, N), jnp.int32),
        ],
    )(coords)
    out_flat = _scatter_mean(features.reshape(B * C * N), idx.reshape(B * N))
    return out_flat.reshape(B, C, R, R, R), norm


# speedup vs baseline: 1.6375x; 1.5544x over previous
import dataclasses

import jax
import jax.numpy as jnp
from jax import lax
from jax.experimental import pallas as pl
from jax.experimental.pallas import tpu as pltpu
from jax.experimental.pallas import tpu_sc as plsc

R = 32
B = 4
C = 128
N = 100000
V = R * R * R
CH = 10000
NCHUNK = N // CH
UNROLL = 5
NIT = CH // (16 * UNROLL)


def _coords_body(coords_ref, norm_ref, idx_ref):
    c = coords_ref[...]
    norm = jnp.clip(((c + 1.0) / 2.0) * R, 0.0, R - 1.0)
    norm_ref[...] = norm
    v = jnp.round(norm).astype(jnp.int32)
    idx_ref[...] = (v[:, 0] * R + v[:, 1]) * R + v[:, 2]


def _sc_body(feat_hbm, idx_hbm, out_hbm, g0, g1, cnt_ref, idx_buf, v0, v1):
    wid = lax.axis_index("s") * 2 + lax.axis_index("c")
    b = wid % B
    g = wid // B
    row0 = b * C + g * 16

    zeros = jnp.zeros((16,), jnp.float32)
    ones = jnp.ones((16,), jnp.float32)

    @plsc.parallel_loop(0, V // 16, unroll=8)
    def _(i):
        cnt_ref[pl.ds(i * 16, 16)] = zeros

    @pl.loop(0, NCHUNK)
    def _(ci):
        pltpu.sync_copy(idx_hbm.at[pl.ds(b * N + ci * CH, CH)], idx_buf)

        @plsc.parallel_loop(0, CH // 16, unroll=UNROLL)
        def _(i):
            iv = idx_buf[pl.ds(i * 16, 16)]
            plsc.addupdate_scatter(cnt_ref, [iv], ones)

    @plsc.parallel_loop(0, V // 16, unroll=4)
    def _(i):
        sl = pl.ds(i * 16, 16)
        cnt_ref[sl] = 1.0 / jnp.maximum(cnt_ref[sl], 1.0)

    @pl.loop(0, 8)
    def _(p):
        rowa = row0 + p * 2

        @plsc.parallel_loop(0, V // 16, unroll=8)
        def _(i):
            sl = pl.ds(i * 16, 16)
            g0[sl] = zeros
            g1[sl] = zeros

        @pl.loop(0, NCHUNK)
        def _(ci):
            off = ci * CH
            pltpu.sync_copy(idx_hbm.at[pl.ds(b * N + off, CH)], idx_buf)
            pltpu.sync_copy(feat_hbm.at[pl.ds(rowa * N + off, CH)], v0)
            pltpu.sync_copy(feat_hbm.at[pl.ds((rowa + 1) * N + off, CH)], v1)

            @plsc.parallel_loop(0, CH // 16, unroll=UNROLL)
            def _(i):
                sl = pl.ds(i * 16, 16)
                iv = idx_buf[sl]
                plsc.addupdate_scatter(g0, [iv], v0[sl])
                plsc.addupdate_scatter(g1, [iv], v1[sl])

        @plsc.parallel_loop(0, V // 16, unroll=4)
        def _(i):
            sl = pl.ds(i * 16, 16)
            g0[sl] = g0[sl] * cnt_ref[sl]
            g1[sl] = g1[sl] * cnt_ref[sl]

        pltpu.sync_copy(g0, out_hbm.at[pl.ds(rowa * V, V)])
        pltpu.sync_copy(g1, out_hbm.at[pl.ds((rowa + 1) * V, V)])


_cp = pltpu.CompilerParams()
if "needs_layout_passes" in pltpu.CompilerParams.__dataclass_fields__:
    _cp = dataclasses.replace(_cp, needs_layout_passes=False)

_scatter_mean = pl.kernel(
    _sc_body,
    out_type=jax.ShapeDtypeStruct((B * C * V,), jnp.float32),
    mesh=plsc.VectorSubcoreMesh(core_axis_name="c", subcore_axis_name="s"),
    scratch_types=[
        pltpu.VMEM((V,), jnp.float32),
        pltpu.VMEM((V,), jnp.float32),
        pltpu.VMEM((V,), jnp.float32),
        pltpu.VMEM((CH,), jnp.int32),
        pltpu.VMEM((CH,), jnp.float32),
        pltpu.VMEM((CH,), jnp.float32),
    ],
    compiler_params=_cp,
)


def kernel(features, coords):
    norm, idx = pl.pallas_call(
        _coords_body,
        out_shape=[
            jax.ShapeDtypeStruct((B, 3, N), jnp.float32),
            jax.ShapeDtypeStruct((B, N), jnp.int32),
        ],
    )(coords)
    out_flat = _scatter_mean(features.reshape(B * C * N), idx.reshape(B * N))
    return out_flat.reshape(B, C, R, R, R), norm


# speedup vs baseline: 1.6427x; 1.0032x over previous
import dataclasses

import jax
import jax.numpy as jnp
from jax import lax
from jax.experimental import pallas as pl
from jax.experimental.pallas import tpu as pltpu
from jax.experimental.pallas import tpu_sc as plsc

R = 32
B = 4
C = 128
N = 100000
V = R * R * R
CH = 10000
NCHUNK = N // CH
UNROLL = 5
NIT = CH // (16 * UNROLL)


def _coords_body(coords_ref, norm_ref, idx_ref):
    c = coords_ref[...]
    norm = jnp.clip(((c + 1.0) / 2.0) * R, 0.0, R - 1.0)
    norm_ref[...] = norm
    v = jnp.round(norm).astype(jnp.int32)
    idx_ref[...] = (v[:, 0] * R + v[:, 1]) * R + v[:, 2]


def _sc_body(feat_hbm, idx_hbm, out_hbm, g0, g1, cnt_ref, idx_buf, v0, v1):
    wid = lax.axis_index("s") * 2 + lax.axis_index("c")
    row0 = wid * 4

    zeros = jnp.zeros((16,), jnp.float32)
    ones = jnp.ones((16,), jnp.float32)

    @plsc.parallel_loop(0, V // 16, unroll=8)
    def _(i):
        cnt_ref[pl.ds(i * 16, 16)] = zeros

    @pl.loop(0, NCHUNK)
    def _(ci):
        pltpu.sync_copy(idx_hbm.at[pl.ds(ci * CH, CH)], idx_buf)

        @plsc.parallel_loop(0, CH // 16, unroll=UNROLL)
        def _(i):
            iv = idx_buf[pl.ds(i * 16, 16)]
            plsc.addupdate_scatter(cnt_ref, [iv], ones)

    @plsc.parallel_loop(0, V // 16, unroll=4)
    def _(i):
        sl = pl.ds(i * 16, 16)
        cnt_ref[sl] = 1.0 / jnp.maximum(cnt_ref[sl], 1.0)

    @pl.loop(0, 2)
    def _(p):
        rowa = row0 + p * 2

        @plsc.parallel_loop(0, V // 16, unroll=8)
        def _(i):
            sl = pl.ds(i * 16, 16)
            g0[sl] = zeros
            g1[sl] = zeros

        @pl.loop(0, NCHUNK)
        def _(ci):
            off = ci * CH
            pltpu.sync_copy(idx_hbm.at[pl.ds(off, CH)], idx_buf)
            pltpu.sync_copy(feat_hbm.at[pl.ds(rowa * N + off, CH)], v0)
            pltpu.sync_copy(feat_hbm.at[pl.ds((rowa + 1) * N + off, CH)], v1)

            @plsc.parallel_loop(0, CH // 16, unroll=UNROLL)
            def _(i):
                sl = pl.ds(i * 16, 16)
                iv = idx_buf[sl]
                plsc.addupdate_scatter(g0, [iv], v0[sl])
                plsc.addupdate_scatter(g1, [iv], v1[sl])

        @plsc.parallel_loop(0, V // 16, unroll=4)
        def _(i):
            sl = pl.ds(i * 16, 16)
            g0[sl] = g0[sl] * cnt_ref[sl]
            g1[sl] = g1[sl] * cnt_ref[sl]

        pltpu.sync_copy(g0, out_hbm.at[pl.ds(rowa * V, V)])
        pltpu.sync_copy(g1, out_hbm.at[pl.ds((rowa + 1) * V, V)])


_cp = pltpu.CompilerParams()
if "needs_layout_passes" in pltpu.CompilerParams.__dataclass_fields__:
    _cp = dataclasses.replace(_cp, needs_layout_passes=False)

_scatter_mean = pl.kernel(
    _sc_body,
    out_type=jax.ShapeDtypeStruct((C * V,), jnp.float32),
    mesh=plsc.VectorSubcoreMesh(core_axis_name="c", subcore_axis_name="s"),
    scratch_types=[
        pltpu.VMEM((V,), jnp.float32),
        pltpu.VMEM((V,), jnp.float32),
        pltpu.VMEM((V,), jnp.float32),
        pltpu.VMEM((CH,), jnp.int32),
        pltpu.VMEM((CH,), jnp.float32),
        pltpu.VMEM((CH,), jnp.float32),
    ],
    compiler_params=_cp,
)


def kernel(features, coords):
    norm, idx = pl.pallas_call(
        _coords_body,
        out_shape=[
            jax.ShapeDtypeStruct((B, 3, N), jnp.float32),
            jax.ShapeDtypeStruct((B, N), jnp.int32),
        ],
    )(coords)
    outs = [
        _scatter_mean(features[q].reshape(C * N), idx[q])
        for q in range(B)
    ]
    out = jnp.stack(outs).reshape(B, C, R, R, R)
    return out, norm


# speedup vs baseline: 1.6796x; 1.0224x over previous
import dataclasses

import jax
import jax.numpy as jnp
from jax import lax
from jax.experimental import pallas as pl
from jax.experimental.pallas import tpu as pltpu
from jax.experimental.pallas import tpu_sc as plsc

R = 32
B = 4
C = 128
N = 100000
V = R * R * R
CH = 1152
NFULL = 86
TAIL = N - NFULL * CH


def _coords_body(coords_ref, norm_ref, idx_ref):
    c = coords_ref[...]
    norm = jnp.clip(((c + 1.0) / 2.0) * R, 0.0, R - 1.0)
    norm_ref[...] = norm
    v = jnp.round(norm).astype(jnp.int32)
    idx_ref[...] = (v[:, 0] * R + v[:, 1]) * R + v[:, 2]


def _sc_body(feat_hbm, idx_hbm, out_hbm, g0, g1, cnt_ref,
             blk0, blk1, tailbuf, ib0, ib1, sem0, sem1):
    wid = lax.axis_index("s") * 2 + lax.axis_index("c")
    r8 = (wid // 2) * 8
    half = (wid % 2) * 4

    zeros = jnp.zeros((16,), jnp.float32)
    ones = jnp.ones((16,), jnp.float32)

    def fire_idx(ci, ib, sem):
        pltpu.async_copy(idx_hbm.at[pl.ds(ci * CH, CH)], ib, sem)

    def drain_idx(ib, sem):
        pltpu.make_async_copy(idx_hbm.at[pl.ds(0, CH)], ib, sem).wait()

    @plsc.parallel_loop(0, V // 16, unroll=8)
    def _(i):
        cnt_ref[pl.ds(i * 16, 16)] = zeros

    def count_from(ib, ngroups, unroll):
        @plsc.parallel_loop(0, ngroups, unroll=unroll)
        def _(i):
            iv = ib[pl.ds(i * 16, 16)]
            plsc.addupdate_scatter(cnt_ref, [iv], ones)

    fire_idx(0, ib0, sem0)

    @pl.loop(0, NFULL, step=2)
    def _(ci):
        fire_idx(ci + 1, ib1, sem1)
        drain_idx(ib0, sem0)
        count_from(ib0, CH // 16, 8)

        @pl.when(ci + 2 < NFULL)
        def _():
            fire_idx(ci + 2, ib0, sem0)

        drain_idx(ib1, sem1)
        count_from(ib1, CH // 16, 8)

    pltpu.async_copy(idx_hbm.at[pl.ds(NFULL * CH, TAIL)],
                     ib0.at[pl.ds(0, TAIL)], sem0).wait()
    count_from(ib0, TAIL // 16, 2)

    @plsc.parallel_loop(0, V // 16, unroll=4)
    def _(i):
        sl = pl.ds(i * 16, 16)
        cnt_ref[sl] = 1.0 / jnp.maximum(cnt_ref[sl], 1.0)

    for p in range(2):
        j0 = half + p * 2
        rowa = r8 + j0

        @plsc.parallel_loop(0, V // 16, unroll=8)
        def _(i):
            sl = pl.ds(i * 16, 16)
            g0[sl] = zeros
            g1[sl] = zeros

        def fire(ci, blk, ib, sem):
            off = ci * CH
            pltpu.async_copy(feat_hbm.at[pl.ds(r8, 8), pl.ds(off, CH)], blk, sem)
            pltpu.async_copy(idx_hbm.at[pl.ds(off, CH)], ib, sem)

        def drain(blk, ib, sem):
            pltpu.make_async_copy(
                feat_hbm.at[pl.ds(0, 8), pl.ds(0, CH)], blk, sem).wait()
            pltpu.make_async_copy(idx_hbm.at[pl.ds(0, CH)], ib, sem).wait()

        def compute(blk, ib, ngroups, unroll):
            @plsc.parallel_loop(0, ngroups, unroll=unroll)
            def _(i):
                sl = pl.ds(i * 16, 16)
                iv = ib[sl]
                plsc.addupdate_scatter(g0, [iv], blk[j0, sl])
                plsc.addupdate_scatter(g1, [iv], blk[j0 + 1, sl])

        fire(0, blk0, ib0, sem0)

        @pl.loop(0, NFULL, step=2)
        def _(ci):
            fire(ci + 1, blk1, ib1, sem1)
            drain(blk0, ib0, sem0)
            compute(blk0, ib0, CH // 16, 8)

            @pl.when(ci + 2 < NFULL)
            def _():
                fire(ci + 2, blk0, ib0, sem0)

            drain(blk1, ib1, sem1)
            compute(blk1, ib1, CH // 16, 8)

        toff = NFULL * CH
        pltpu.async_copy(feat_hbm.at[pl.ds(r8, 8), pl.ds(toff, TAIL)],
                         tailbuf, sem0).wait()
        pltpu.async_copy(idx_hbm.at[pl.ds(toff, TAIL)],
                         ib0.at[pl.ds(0, TAIL)], sem0).wait()
        compute(tailbuf, ib0, TAIL // 16, 2)

        @plsc.parallel_loop(0, V // 16, unroll=4)
        def _(i):
            sl = pl.ds(i * 16, 16)
            g0[sl] = g0[sl] * cnt_ref[sl]
            g1[sl] = g1[sl] * cnt_ref[sl]

        pltpu.sync_copy(g0, out_hbm.at[pl.ds(rowa * V, V)])
        pltpu.sync_copy(g1, out_hbm.at[pl.ds((rowa + 1) * V, V)])


_cp = pltpu.CompilerParams()
if "needs_layout_passes" in pltpu.CompilerParams.__dataclass_fields__:
    _cp = dataclasses.replace(_cp, needs_layout_passes=False)

_scatter_mean = pl.kernel(
    _sc_body,
    out_type=jax.ShapeDtypeStruct((C * V,), jnp.float32),
    mesh=plsc.VectorSubcoreMesh(core_axis_name="c", subcore_axis_name="s"),
    scratch_types=[
        pltpu.VMEM((V,), jnp.float32),
        pltpu.VMEM((V,), jnp.float32),
        pltpu.VMEM((V,), jnp.float32),
        pltpu.VMEM((8, CH), jnp.float32),
        pltpu.VMEM((8, CH), jnp.float32),
        pltpu.VMEM((8, TAIL), jnp.float32),
        pltpu.VMEM((CH,), jnp.int32),
        pltpu.VMEM((CH,), jnp.int32),
        pltpu.SemaphoreType.DMA,
        pltpu.SemaphoreType.DMA,
    ],
    compiler_params=_cp,
)


def kernel(features, coords):
    norm, idx = pl.pallas_call(
        _coords_body,
        out_shape=[
            jax.ShapeDtypeStruct((B, 3, N), jnp.float32),
            jax.ShapeDtypeStruct((B, N), jnp.int32),
        ],
    )(coords)
    outs = [_scatter_mean(features[q], idx[q]) for q in range(B)]
    out = jnp.stack(outs).reshape(B, C, R, R, R)
    return out, norm


# speedup vs baseline: 1.8686x; 1.1126x over previous
import dataclasses

import jax
import jax.numpy as jnp
from jax import lax
from jax.experimental import pallas as pl
from jax.experimental.pallas import tpu as pltpu
from jax.experimental.pallas import tpu_sc as plsc

R = 32
B = 4
C = 128
N = 100000
V = R * R * R
CH = 4000
NCHUNK = N // CH


def _coords_body(coords_ref, norm_ref, idx_ref):
    c = coords_ref[...]
    norm = jnp.clip(((c + 1.0) / 2.0) * R, 0.0, R - 1.0)
    norm_ref[...] = norm
    v = jnp.round(norm).astype(jnp.int32)
    idx_ref[...] = (v[:, 0] * R + v[:, 1]) * R + v[:, 2]


def _make_sc_body(q):
    def _sc_body(feat_hbm, idx_hbm, out_hbm, g0, g1, cnt_ref,
                 va0, vb0, va1, vb1, ia0, ia1, sem0, sem1):
        wid = lax.axis_index("s") * 2 + lax.axis_index("c")
        row0 = wid * 4

        zeros = jnp.zeros((16,), jnp.float32)
        ones = jnp.ones((16,), jnp.float32)
        ibase = q * N

        def fire_idx(ci, ib, sem):
            pltpu.async_copy(idx_hbm.at[pl.ds(ibase + ci * CH, CH)], ib, sem)

        def drain_idx(ib, sem):
            pltpu.make_async_copy(idx_hbm.at[pl.ds(0, CH)], ib, sem).wait()

        @plsc.parallel_loop(0, V // 16, unroll=8)
        def _(i):
            cnt_ref[pl.ds(i * 16, 16)] = zeros

        def count_from(ib):
            @plsc.parallel_loop(0, CH // 16, unroll=5)
            def _(i):
                iv = ib[pl.ds(i * 16, 16)]
                plsc.addupdate_scatter(cnt_ref, [iv], ones)

        fire_idx(0, ia0, sem0)

        @pl.loop(0, NCHUNK - 1, step=2)
        def _(ci):
            fire_idx(ci + 1, ia1, sem1)
            drain_idx(ia0, sem0)
            count_from(ia0)
            fire_idx(ci + 2, ia0, sem0)
            drain_idx(ia1, sem1)
            count_from(ia1)

        drain_idx(ia0, sem0)
        count_from(ia0)

        @plsc.parallel_loop(0, V // 16, unroll=4)
        def _(i):
            sl = pl.ds(i * 16, 16)
            cnt_ref[sl] = 1.0 / jnp.maximum(cnt_ref[sl], 1.0)

        for p in range(2):
            rowa = row0 + p * 2
            abase = (q * C + rowa) * N
            bbase = abase + N

            @plsc.parallel_loop(0, V // 16, unroll=8)
            def _(i):
                sl = pl.ds(i * 16, 16)
                g0[sl] = zeros
                g1[sl] = zeros

            def fire(ci, va, vb, ib, sem):
                off = ci * CH
                pltpu.async_copy(feat_hbm.at[pl.ds(abase + off, CH)], va, sem)
                pltpu.async_copy(feat_hbm.at[pl.ds(bbase + off, CH)], vb, sem)
                pltpu.async_copy(idx_hbm.at[pl.ds(ibase + off, CH)], ib, sem)

            def drain(va, vb, ib, sem):
                pltpu.make_async_copy(
                    feat_hbm.at[pl.ds(0, CH)], va, sem).wait()
                pltpu.make_async_copy(
                    feat_hbm.at[pl.ds(0, CH)], vb, sem).wait()
                pltpu.make_async_copy(idx_hbm.at[pl.ds(0, CH)], ib, sem).wait()

            def compute(va, vb, ib):
                @plsc.parallel_loop(0, CH // 16, unroll=5)
                def _(i):
                    sl = pl.ds(i * 16, 16)
                    iv = ib[sl]
                    plsc.addupdate_scatter(g0, [iv], va[sl])
                    plsc.addupdate_scatter(g1, [iv], vb[sl])

            fire(0, va0, vb0, ia0, sem0)

            @pl.loop(0, NCHUNK - 1, step=2)
            def _(ci):
                fire(ci + 1, va1, vb1, ia1, sem1)
                drain(va0, vb0, ia0, sem0)
                compute(va0, vb0, ia0)
                fire(ci + 2, va0, vb0, ia0, sem0)
                drain(va1, vb1, ia1, sem1)
                compute(va1, vb1, ia1)

            drain(va0, vb0, ia0, sem0)
            compute(va0, vb0, ia0)

            @plsc.parallel_loop(0, V // 16, unroll=4)
            def _(i):
                sl = pl.ds(i * 16, 16)
                g0[sl] = g0[sl] * cnt_ref[sl]
                g1[sl] = g1[sl] * cnt_ref[sl]

            pltpu.sync_copy(g0, out_hbm.at[pl.ds(rowa * V, V)])
            pltpu.sync_copy(g1, out_hbm.at[pl.ds((rowa + 1) * V, V)])

    return _sc_body


_cp = pltpu.CompilerParams()
if "needs_layout_passes" in pltpu.CompilerParams.__dataclass_fields__:
    _cp = dataclasses.replace(_cp, needs_layout_passes=False)

_sc_calls = [
    pl.kernel(
        _make_sc_body(q),
        out_type=jax.ShapeDtypeStruct((C * V,), jnp.float32),
        mesh=plsc.VectorSubcoreMesh(core_axis_name="c", subcore_axis_name="s"),
        scratch_types=[
            pltpu.VMEM((V,), jnp.float32),
            pltpu.VMEM((V,), jnp.float32),
            pltpu.VMEM((V,), jnp.float32),
            pltpu.VMEM((CH,), jnp.float32),
            pltpu.VMEM((CH,), jnp.float32),
            pltpu.VMEM((CH,), jnp.float32),
            pltpu.VMEM((CH,), jnp.float32),
            pltpu.VMEM((CH,), jnp.int32),
            pltpu.VMEM((CH,), jnp.int32),
            pltpu.SemaphoreType.DMA,
            pltpu.SemaphoreType.DMA,
        ],
        compiler_params=_cp,
    )
    for q in range(B)
]


def kernel(features, coords):
    norm, idx = pl.pallas_call(
        _coords_body,
        out_shape=[
            jax.ShapeDtypeStruct((B, 3, N), jnp.float32),
            jax.ShapeDtypeStruct((B, N), jnp.int32),
        ],
    )(coords)
    feat_flat = features.reshape(B * C * N)
    idx_flat = idx.reshape(B * N)
    outs = [_sc_calls[q](feat_flat, idx_flat) for q in range(B)]
    out = jnp.stack(outs).reshape(B, C, R, R, R)
    return out, norm


# speedup vs baseline: 2.0573x; 1.1010x over previous
import dataclasses

import jax
import jax.numpy as jnp
from jax import lax
from jax.experimental import pallas as pl
from jax.experimental.pallas import tpu as pltpu
from jax.experimental.pallas import tpu_sc as plsc

R = 32
B = 4
C = 128
N = 100000
V = R * R * R
CH = 4000
NCHUNK = N // CH


def _coords_body(coords_ref, norm_ref, idx_ref):
    c = coords_ref[...]
    norm = jnp.clip(((c + 1.0) / 2.0) * R, 0.0, R - 1.0)
    norm_ref[...] = norm
    v = jnp.round(norm).astype(jnp.int32)
    idx_ref[...] = (v[:, 0] * R + v[:, 1]) * R + v[:, 2]


def _make_sc_body(q):
    def _sc_body(feat_hbm, idx_hbm, out_hbm, g0, g1, cnt_ref,
                 va0, vb0, va1, vb1, ia0, ia1, sem0, sem1):
        wid = lax.axis_index("s") * 2 + lax.axis_index("c")
        row0 = wid * 4

        zeros = jnp.zeros((16,), jnp.float32)
        ones = jnp.ones((16,), jnp.float32)
        ibase = q * N

        def fire_idx(ci, ib, sem):
            pltpu.async_copy(idx_hbm.at[pl.ds(ibase + ci * CH, CH)], ib, sem)

        def drain_idx(ib, sem):
            pltpu.make_async_copy(idx_hbm.at[pl.ds(0, CH)], ib, sem).wait()

        @plsc.parallel_loop(0, V // 16, unroll=8)
        def _(i):
            cnt_ref[pl.ds(i * 16, 16)] = zeros

        def count_from(ib):
            @plsc.parallel_loop(0, CH // 16, unroll=5)
            def _(i):
                iv = ib[pl.ds(i * 16, 16)]
                plsc.addupdate_scatter(cnt_ref, [iv], ones)

        fire_idx(0, ia0, sem0)

        @pl.loop(0, NCHUNK - 1, step=2)
        def _(ci):
            fire_idx(ci + 1, ia1, sem1)
            drain_idx(ia0, sem0)
            count_from(ia0)
            fire_idx(ci + 2, ia0, sem0)
            drain_idx(ia1, sem1)
            count_from(ia1)

        drain_idx(ia0, sem0)
        count_from(ia0)

        @plsc.parallel_loop(0, V // 16, unroll=4)
        def _(i):
            sl = pl.ds(i * 16, 16)
            cnt_ref[sl] = 1.0 / jnp.maximum(cnt_ref[sl], 1.0)

        for p in range(2):
            rowa = row0 + p * 2
            abase = (q * C + rowa) * N
            bbase = abase + N

            @plsc.parallel_loop(0, V // 16, unroll=8)
            def _(i):
                sl = pl.ds(i * 16, 16)
                g0[sl] = zeros
                g1[sl] = zeros

            def fire(ci, va, vb, ib, sem):
                off = ci * CH
                pltpu.async_copy(feat_hbm.at[pl.ds(abase + off, CH)], va, sem)
                pltpu.async_copy(feat_hbm.at[pl.ds(bbase + off, CH)], vb, sem)
                pltpu.async_copy(idx_hbm.at[pl.ds(ibase + off, CH)], ib, sem)

            def drain(va, vb, ib, sem):
                pltpu.make_async_copy(
                    feat_hbm.at[pl.ds(0, CH)], va, sem).wait()
                pltpu.make_async_copy(
                    feat_hbm.at[pl.ds(0, CH)], vb, sem).wait()
                pltpu.make_async_copy(idx_hbm.at[pl.ds(0, CH)], ib, sem).wait()

            def compute(va, vb, ib):
                @plsc.parallel_loop(0, CH // 16, unroll=5)
                def _(i):
                    sl = pl.ds(i * 16, 16)
                    iv = ib[sl]
                    plsc.addupdate_scatter(g0, [iv], va[sl])
                    plsc.addupdate_scatter(g1, [iv], vb[sl])

            fire(0, va0, vb0, ia0, sem0)

            @pl.loop(0, NCHUNK - 1, step=2)
            def _(ci):
                fire(ci + 1, va1, vb1, ia1, sem1)
                drain(va0, vb0, ia0, sem0)
                compute(va0, vb0, ia0)
                fire(ci + 2, va0, vb0, ia0, sem0)
                drain(va1, vb1, ia1, sem1)
                compute(va1, vb1, ia1)

            drain(va0, vb0, ia0, sem0)
            compute(va0, vb0, ia0)

            @plsc.parallel_loop(0, V // 16, unroll=4)
            def _(i):
                sl = pl.ds(i * 16, 16)
                g0[sl] = g0[sl] * cnt_ref[sl]
                g1[sl] = g1[sl] * cnt_ref[sl]

            pltpu.sync_copy(g0, out_hbm.at[pl.ds(rowa * V, V)])
            pltpu.sync_copy(g1, out_hbm.at[pl.ds((rowa + 1) * V, V)])

    return _sc_body


_cp = pltpu.CompilerParams()
if "needs_layout_passes" in pltpu.CompilerParams.__dataclass_fields__:
    _cp = dataclasses.replace(_cp, needs_layout_passes=False)

_sc_calls = [
    pl.kernel(
        _make_sc_body(0),
        out_type=jax.ShapeDtypeStruct((C * V,), jnp.float32),
        mesh=plsc.VectorSubcoreMesh(core_axis_name="c", subcore_axis_name="s"),
        scratch_types=[
            pltpu.VMEM((V,), jnp.float32),
            pltpu.VMEM((V,), jnp.float32),
            pltpu.VMEM((V,), jnp.float32),
            pltpu.VMEM((CH,), jnp.float32),
            pltpu.VMEM((CH,), jnp.float32),
            pltpu.VMEM((CH,), jnp.float32),
            pltpu.VMEM((CH,), jnp.float32),
            pltpu.VMEM((CH,), jnp.int32),
            pltpu.VMEM((CH,), jnp.int32),
            pltpu.SemaphoreType.DMA,
            pltpu.SemaphoreType.DMA,
        ],
        compiler_params=_cp,
    )
    for q in range(B)
]


def kernel(features, coords):
    norm, idx = pl.pallas_call(
        _coords_body,
        out_shape=[
            jax.ShapeDtypeStruct((B, 3, N), jnp.float32),
            jax.ShapeDtypeStruct((B, N), jnp.int32),
        ],
    )(coords)
    outs = [
        _sc_calls[q](features[q].reshape(C * N), idx[q]) for q in range(B)
    ]
    out = jnp.stack(outs).reshape(B, C, R, R, R)
    return out, norm
